# baseline (speedup 1.0000x reference)
import jax
import jax.numpy as jnp
from jax import lax
from jax.experimental import pallas as pl
from jax.experimental.pallas import tpu as pltpu

_VOCAB = 32000
_LANES = 128
_R = 250
_BT = 128


def _gather_block_kernel(ids_ref, means_ref, out_ref):
    bt = ids_ref.shape[0]
    ids = jnp.clip(ids_ref[...], 0, _VOCAB - 1)
    row = lax.shift_right_logical(ids, 7)
    lane = jnp.bitwise_and(ids, _LANES - 1)
    row_bf = row.astype(jnp.bfloat16)
    lane_bf = lane.astype(jnp.bfloat16)

    means_bf = means_ref[...].astype(jnp.bfloat16)

    one = jnp.bfloat16(1.0)
    zero = jnp.bfloat16(0.0)
    oh_iota = lax.broadcasted_iota(
        jnp.int32, (_R, _LANES), 0).astype(jnp.bfloat16)
    oh_pieces = [
        jnp.where(oh_iota == row_bf[i:i + 1, :], one, zero) for i in range(bt)
    ]
    onehot = jnp.concatenate(oh_pieces, axis=1)

    rowvals = lax.dot_general(
        means_bf, onehot,
        (((0,), (0,)), ((), ())),
        preferred_element_type=jnp.float32)

    sel_iota = lax.broadcasted_iota(
        jnp.int32, (_LANES, _LANES), 0).astype(jnp.bfloat16)
    lane_mask = jnp.concatenate(
        [sel_iota == lane_bf[i:i + 1, :] for i in range(bt)], axis=1)
    picked = jnp.where(lane_mask, rowvals.astype(jnp.bfloat16),
                       jnp.bfloat16(0.0))

    y = jnp.sum(picked.astype(jnp.float32), axis=0,
                keepdims=True)
    sub_iota = lax.broadcasted_iota(jnp.int32, (8, _LANES), 0)
    for g in range(bt // 8):
        tile = jnp.where(sub_iota == 0,
                         y[:, (8 * g) * _LANES:(8 * g + 1) * _LANES], 0.0)
        for i in range(1, 8):
            c = 8 * g + i
            tile = tile + jnp.where(
                sub_iota == i, y[:, c * _LANES:(c + 1) * _LANES], 0.0)
        out_ref[8 * g:8 * g + 8, :] = tile


def kernel(dev_id, dev_mask, rom_id, rom_mask, emb_table, row_means):
    del dev_mask, rom_id, rom_mask, emb_table
    B, L = dev_id.shape
    assert L == _LANES and B % _BT == 0

    return pl.pallas_call(
        _gather_block_kernel,
        out_shape=jax.ShapeDtypeStruct((B, L), jnp.float32),
        grid=(B // _BT,),
        in_specs=[
            pl.BlockSpec((_BT, _LANES), lambda i: (i, 0)),
            pl.BlockSpec((_R, _LANES), lambda i: (0, 0)),
        ],
        out_specs=pl.BlockSpec((_BT, _LANES), lambda i: (i, 0)),
        compiler_params=pltpu.CompilerParams(
            dimension_semantics=("parallel",),
            vmem_limit_bytes=56 * 1024 * 1024),
    )(dev_id.astype(jnp.int32), row_means)


# speedup vs baseline: 1.1298x; 1.1298x over previous
import jax
import jax.numpy as jnp
from jax import lax
from jax.experimental import pallas as pl
from jax.experimental.pallas import tpu as pltpu

_VOCAB = 32000
_LANES = 128
_R = 250
_BT = 128


def _gather_block_kernel(ids_ref, means_ref, out_ref):
    bt = ids_ref.shape[0]
    ids = jnp.clip(ids_ref[...], 0, _VOCAB - 1)
    row = lax.shift_right_logical(ids, 7)
    lane = jnp.bitwise_and(ids, _LANES - 1)
    row_bf = row.astype(jnp.bfloat16)
    lane_bf = lane.astype(jnp.bfloat16)

    means_bf = means_ref[...].astype(jnp.bfloat16)

    one = jnp.bfloat16(1.0)
    zero = jnp.bfloat16(0.0)
    oh_iota = lax.broadcasted_iota(
        jnp.int32, (_R, _LANES), 0).astype(jnp.bfloat16)
    oh_pieces = [
        jnp.where(oh_iota == row_bf[i:i + 1, :], one, zero) for i in range(bt)
    ]
    onehot = jnp.concatenate(oh_pieces, axis=1)

    rowvals = lax.dot_general(
        means_bf, onehot,
        (((0,), (0,)), ((), ())),
        preferred_element_type=jnp.float32)

    sel_iota = lax.broadcasted_iota(
        jnp.int32, (_LANES, _LANES), 0).astype(jnp.bfloat16)
    lane_mask = jnp.concatenate(
        [sel_iota == lane_bf[i:i + 1, :] for i in range(bt)], axis=1)
    picked = jnp.where(lane_mask, rowvals.astype(jnp.bfloat16),
                       jnp.bfloat16(0.0))

    ones8 = jnp.full((8, _LANES), 1.0, jnp.bfloat16)
    y = lax.dot_general(
        ones8, picked,
        (((1,), (0,)), ((), ())),
        preferred_element_type=jnp.float32)
    sub_iota = lax.broadcasted_iota(jnp.int32, (8, _LANES), 0)
    for g in range(bt // 8):
        tile = jnp.where(sub_iota == 0,
                         y[:, (8 * g) * _LANES:(8 * g + 1) * _LANES], 0.0)
        for i in range(1, 8):
            c = 8 * g + i
            tile = tile + jnp.where(
                sub_iota == i, y[:, c * _LANES:(c + 1) * _LANES], 0.0)
        out_ref[8 * g:8 * g + 8, :] = tile


def kernel(dev_id, dev_mask, rom_id, rom_mask, emb_table, row_means):
    del dev_mask, rom_id, rom_mask, emb_table
    B, L = dev_id.shape
    assert L == _LANES and B % _BT == 0

    return pl.pallas_call(
        _gather_block_kernel,
        out_shape=jax.ShapeDtypeStruct((B, L), jnp.float32),
        grid=(B // _BT,),
        in_specs=[
            pl.BlockSpec((_BT, _LANES), lambda i: (i, 0)),
            pl.BlockSpec((_R, _LANES), lambda i: (0, 0)),
        ],
        out_specs=pl.BlockSpec((_BT, _LANES), lambda i: (i, 0)),
        compiler_params=pltpu.CompilerParams(
            dimension_semantics=("parallel",),
            vmem_limit_bytes=56 * 1024 * 1024),
    )(dev_id.astype(jnp.int32), row_means)
